# Initial kernel scaffold; baseline (speedup 1.0000x reference)
import jax
import jax.numpy as jnp
from jax.experimental import pallas as pl


def kernel(x, edge_index, batch, W1, b1, W2, b2, W3, b3, W4, b4):
    raise NotImplementedError("write your pallas kernel here")


# speedup vs baseline: 15.8722x; 15.8722x over previous
import functools

import jax
import jax.numpy as jnp
from jax import lax
from jax.experimental import pallas as pl
from jax.experimental.pallas import tpu as pltpu
from jax.experimental.pallas import tpu_sc as plsc

N = 10000
E = 320000
D_IN = 128
H = 64
G = 16

NC = 2
NS = 16
NW = NC * NS
CH = 128
NCHUNK = -(-E // (NW * CH))
E_PAD = NW * NCHUNK * CH
N_PAD = 10112
RPT = N_PAD // NS


def _mesh():
    return plsc.VectorSubcoreMesh(
        core_axis_name="c", subcore_axis_name="s", num_cores=NC, num_subcores=NS
    )


_SC_PARAMS = pltpu.CompilerParams(use_tc_tiling_on_sc=False)


def _sc_deg_body(dst_hbm, ones_hbm, zeros_hbm, deg_out, dst_v, ones_v, acc_sh, sem):
    c = lax.axis_index("c")
    s = lax.axis_index("s")
    wid = c * NS + s
    pltpu.sync_copy(zeros_hbm.at[pl.ds(s * RPT, RPT)], acc_sh.at[pl.ds(s * RPT, RPT)])
    pltpu.sync_copy(ones_hbm, ones_v)
    pltpu.sync_copy(dst_hbm.at[wid], dst_v)
    plsc.subcore_barrier()

    def chunk(j, carry):
        pltpu.sync_copy(ones_v, acc_sh.at[dst_v.at[j]], add=True)
        return carry

    lax.fori_loop(0, NCHUNK, chunk, 0)
    plsc.subcore_barrier()
    pltpu.sync_copy(
        acc_sh.at[pl.ds(s * RPT, RPT)], deg_out.at[c, pl.ds(s * RPT, RPT)]
    )


def _sc_deg(dst3, ones8, zeros8):
    return pl.kernel(
        _sc_deg_body,
        out_type=jax.ShapeDtypeStruct((NC, N_PAD, 8), jnp.float32),
        mesh=_mesh(),
        scratch_types=[
            pltpu.VMEM((NCHUNK, CH), jnp.int32),
            pltpu.VMEM((CH, 8), jnp.float32),
            pltpu.VMEM_SHARED((N_PAD, 8), jnp.float32),
            pltpu.SemaphoreType.DMA,
        ],
        compiler_params=_SC_PARAMS,
    )(dst3, ones8, zeros8)


def _sc_scatter_body(z_hbm, src_hbm, dst_hbm, zeros_hbm, acc_out,
                     src_v, dst_v, rows_v, acc_sh, sem):
    c = lax.axis_index("c")
    s = lax.axis_index("s")
    wid = c * NS + s
    pltpu.sync_copy(zeros_hbm.at[pl.ds(s * RPT, RPT)], acc_sh.at[pl.ds(s * RPT, RPT)])
    pltpu.sync_copy(src_hbm.at[wid], src_v)
    pltpu.sync_copy(dst_hbm.at[wid], dst_v)
    plsc.subcore_barrier()

    def chunk(j, carry):
        pltpu.async_copy(z_hbm.at[src_v.at[j]], rows_v, sem).wait()
        pltpu.sync_copy(rows_v, acc_sh.at[dst_v.at[j]], add=True)
        return carry

    lax.fori_loop(0, NCHUNK, chunk, 0)
    plsc.subcore_barrier()
    pltpu.sync_copy(
        acc_sh.at[pl.ds(s * RPT, RPT)], acc_out.at[c, pl.ds(s * RPT, RPT)]
    )


def _sc_scatter(z_pad, src3, dst3, zeros64):
    return pl.kernel(
        _sc_scatter_body,
        out_type=jax.ShapeDtypeStruct((NC, N_PAD, H), jnp.float32),
        mesh=_mesh(),
        scratch_types=[
            pltpu.VMEM((NCHUNK, CH), jnp.int32),
            pltpu.VMEM((NCHUNK, CH), jnp.int32),
            pltpu.VMEM((CH, H), jnp.float32),
            pltpu.VMEM_SHARED((N_PAD, H), jnp.float32),
            pltpu.SemaphoreType.DMA,
        ],
        compiler_params=_SC_PARAMS,
    )(z_pad, src3, dst3, zeros64)


def _tc1_body(x_ref, w_ref, degp_ref, z_ref, dinv_ref):
    deg = 1.0 + degp_ref[0, 0:N, 0] + degp_ref[1, 0:N, 0]
    dinv = lax.rsqrt(deg)[:, None]
    h = jnp.dot(x_ref[...], w_ref[...], preferred_element_type=jnp.float32)
    z_ref[0:N, :] = dinv * h
    z_ref[N:N_PAD, :] = jnp.zeros((N_PAD - N, H), jnp.float32)
    dinv_ref[0:N, :] = dinv
    dinv_ref[N:N_PAD, :] = jnp.zeros((N_PAD - N, 1), jnp.float32)


def _tc1(x, W1, degp):
    return pl.pallas_call(
        _tc1_body,
        out_shape=(
            jax.ShapeDtypeStruct((N_PAD, H), jnp.float32),
            jax.ShapeDtypeStruct((N_PAD, 1), jnp.float32),
        ),
    )(x, W1, degp)


def _tcmid_body(acc_ref, z_ref, dinv_ref, b_ref, w_ref, zo_ref):
    agg = acc_ref[0, 0:N, :] + acc_ref[1, 0:N, :] + z_ref[0:N, :]
    dinv = dinv_ref[0:N, :]
    h = jnp.maximum(dinv * agg + b_ref[...], 0.0)
    zo_ref[0:N, :] = dinv * jnp.dot(h, w_ref[...], preferred_element_type=jnp.float32)
    zo_ref[N:N_PAD, :] = jnp.zeros((N_PAD - N, H), jnp.float32)


def _tcmid(acc, z, dinv, b_prev, W_next):
    return pl.pallas_call(
        _tcmid_body,
        out_shape=jax.ShapeDtypeStruct((N_PAD, H), jnp.float32),
    )(acc, z, dinv, b_prev, W_next)


def _tc5_body(acc_ref, z_ref, dinv_ref, b_ref, batch_ref, out_ref):
    agg = acc_ref[0, 0:N, :] + acc_ref[1, 0:N, :] + z_ref[0:N, :]
    h = jnp.maximum(dinv_ref[0:N, :] * agg + b_ref[...], 0.0)
    iota = lax.broadcasted_iota(jnp.int32, (N, G), 1)
    onehot = (batch_ref[...] == iota).astype(jnp.float32)
    sums = lax.dot_general(
        onehot, h, (((0,), (0,)), ((), ())), preferred_element_type=jnp.float32
    )
    counts = jnp.sum(onehot, axis=0)
    out_ref[...] = sums / jnp.maximum(counts, 1.0)[:, None]


def _tc5(acc, z, dinv, b4, batch2d):
    return pl.pallas_call(
        _tc5_body,
        out_shape=jax.ShapeDtypeStruct((G, H), jnp.float32),
    )(acc, z, dinv, b4, batch2d)


def kernel(x, edge_index, batch, W1, b1, W2, b2, W3, b3, W4, b4):
    pad = jnp.full((E_PAD - E,), N, jnp.int32)
    src3 = jnp.concatenate([edge_index[0], pad]).reshape(NW, NCHUNK, CH)
    dst3 = jnp.concatenate([edge_index[1], pad]).reshape(NW, NCHUNK, CH)
    ones8 = jnp.ones((CH, 8), jnp.float32)
    zeros8 = jnp.zeros((N_PAD, 8), jnp.float32)
    zeros64 = jnp.zeros((N_PAD, H), jnp.float32)

    degp = _sc_deg(dst3, ones8, zeros8)
    z, dinv = _tc1(x, W1, degp)

    acc = _sc_scatter(z, src3, dst3, zeros64)
    z = _tcmid(acc, z, dinv, b1.reshape(1, H), W2)
    acc = _sc_scatter(z, src3, dst3, zeros64)
    z = _tcmid(acc, z, dinv, b2.reshape(1, H), W3)
    acc = _sc_scatter(z, src3, dst3, zeros64)
    z = _tcmid(acc, z, dinv, b3.reshape(1, H), W4)
    acc = _sc_scatter(z, src3, dst3, zeros64)

    return _tc5(acc, z, dinv, b4.reshape(1, H), batch.reshape(N, 1))
